# baseline (speedup 1.0000x reference)
import jax
import jax.numpy as jnp
from jax.experimental import pallas as pl
from jax.experimental.pallas import tpu as pltpu

HIDDEN = 4096
FF = 2048
E = 64
TOP_K = 8
THRESHOLD = 0.2
TBLK = 512
_BASE_BITS = 1045220557


def _router_kernel(x_ref, w1_hbm, b1_ref, w2_ref, b2_ref,
                   logits_ref, mask_ref, normw_ref, loss_ref,
                   w1_ref, lbuf, acc_mask, acc_prob, w1_sem):
    i = pl.program_id(0)
    n = pl.num_programs(0)

    @pl.when(i == 0)
    def _load_w1():
        pltpu.make_async_copy(w1_hbm, w1_ref, w1_sem).start()
        pltpu.make_async_copy(w1_hbm, w1_ref, w1_sem).wait()

    logits_p = lbuf[...]
    gates = jax.nn.sigmoid(logits_p)
    pos = gates > THRESHOLD

    bits = jax.lax.bitcast_convert_type(gates, jnp.int32)
    eidx = jax.lax.broadcasted_iota(jnp.int32, gates.shape, 1)
    key = ((bits - _BASE_BITS) << 6) | ((E - 1) - eidx)
    cur = jnp.where(pos, key, -1)
    sel = jnp.zeros(gates.shape, dtype=jnp.bool_)
    for _ in range(TOP_K):
        m = jnp.max(cur, axis=1, keepdims=True)
        hit = cur == m
        sel = sel | hit
        cur = jnp.where(hit, -1, cur)
    maskf = jnp.where(sel & pos, 1.0, 0.0)
    mask_ref[...] = maskf

    w = gates * maskf
    normw_ref[...] = w / (jnp.sum(w, axis=1, keepdims=True) + 1e-6)

    mx = jnp.max(logits_p, axis=1, keepdims=True)
    ex = jnp.exp(logits_p - mx)
    probs = ex / jnp.sum(ex, axis=1, keepdims=True)

    mm = jnp.sum(maskf, axis=0, keepdims=True)
    pm = jnp.sum(probs, axis=0, keepdims=True)

    @pl.when(i == 1)
    def _init():
        acc_mask[...] = mm
        acc_prob[...] = pm

    @pl.when(i > 1)
    def _acc():
        acc_mask[...] += mm
        acc_prob[...] += pm

    @pl.when(i == n - 1)
    def _fin():
        t = jnp.float32((n - 1) * TBLK)
        s = jnp.sum(acc_mask[...] * acc_prob[...], axis=1, keepdims=True)
        loss_ref[...] = s * jnp.float32(E) / (t * t)

    x = x_ref[...].astype(jnp.bfloat16)
    h = jax.lax.dot_general(x, w1_ref[...], (((1,), (0,)), ((), ())),
                            preferred_element_type=jnp.float32,
                            precision=jax.lax.Precision.DEFAULT)
    h = jnp.maximum(h + b1_ref[...], 0.0)
    logits = jax.lax.dot_general(h, w2_ref[...], (((1,), (0,)), ((), ())),
                                 preferred_element_type=jnp.float32,
                                 precision=jax.lax.Precision.DEFAULT)
    logits = logits + b2_ref[...]
    logits_ref[...] = logits
    lbuf[...] = logits


def kernel(hidden_states, W1, b1, W2, b2):
    B, S, H = hidden_states.shape
    T = B * S
    x = hidden_states.reshape(T, H)
    w1t = W1.T.astype(jnp.bfloat16)
    w2t = W2.T
    nblk = T // TBLK
    grid = (nblk + 1,)
    last = nblk - 1

    out_shape = [
        jax.ShapeDtypeStruct((T, E), jnp.float32),
        jax.ShapeDtypeStruct((T, E), jnp.float32),
        jax.ShapeDtypeStruct((T, E), jnp.float32),
        jax.ShapeDtypeStruct((1, 1), jnp.float32),
    ]
    in_specs = [
        pl.BlockSpec((TBLK, H), lambda i: (jnp.minimum(i, last), 0)),
        pl.BlockSpec(memory_space=pltpu.MemorySpace.HBM),
        pl.BlockSpec((1, FF), lambda i: (0, 0)),
        pl.BlockSpec((FF, E), lambda i: (0, 0)),
        pl.BlockSpec((1, E), lambda i: (0, 0)),
    ]
    out_specs = [
        pl.BlockSpec((TBLK, E), lambda i: (jnp.minimum(i, last), 0)),
        pl.BlockSpec((TBLK, E), lambda i: (jnp.maximum(i - 1, 0), 0)),
        pl.BlockSpec((TBLK, E), lambda i: (jnp.maximum(i - 1, 0), 0)),
        pl.BlockSpec((1, 1), lambda i: (0, 0)),
    ]
    logits, maskf, normw, loss = pl.pallas_call(
        _router_kernel,
        grid=grid,
        in_specs=in_specs,
        out_specs=out_specs,
        out_shape=out_shape,
        scratch_shapes=[pltpu.VMEM((HIDDEN, FF), jnp.bfloat16),
                        pltpu.VMEM((TBLK, E), jnp.float32),
                        pltpu.VMEM((1, E), jnp.float32),
                        pltpu.VMEM((1, E), jnp.float32),
                        pltpu.SemaphoreType.DMA],
        compiler_params=pltpu.CompilerParams(
            dimension_semantics=("arbitrary",)),
    )(x, w1t, b1.reshape(1, FF), w2t, b2.reshape(1, E))

    dispatch_mask = maskf.astype(bool).reshape(B, S, E)
    normalized_weights = normw.reshape(B, S, E)
    router_logits = logits.reshape(B, S, E)
    return dispatch_mask, normalized_weights, loss[0, 0], router_logits


# speedup vs baseline: 1.0907x; 1.0907x over previous
import jax
import jax.numpy as jnp
from jax.experimental import pallas as pl
from jax.experimental.pallas import tpu as pltpu

HIDDEN = 4096
FF = 2048
E = 64
TOP_K = 8
THRESHOLD = 0.2
TBLK = 512
_BASE_BITS = 1045220557


def _router_kernel(x_ref, w1_hbm, b1_ref, w2_ref, b2_ref,
                   logits_ref, mask_ref, normw_ref, loss_ref,
                   w1_ref, acc_mask, acc_prob, w1_sem):
    i = pl.program_id(0)
    n = pl.num_programs(0)

    @pl.when(i == 0)
    def _load_w1():
        pltpu.make_async_copy(w1_hbm, w1_ref, w1_sem).start()
        pltpu.make_async_copy(w1_hbm, w1_ref, w1_sem).wait()

    x = x_ref[...].astype(jnp.bfloat16)
    h = jax.lax.dot_general(x, w1_ref[...], (((1,), (0,)), ((), ())),
                            preferred_element_type=jnp.float32,
                            precision=jax.lax.Precision.DEFAULT)
    h = jnp.maximum(h + b1_ref[...], 0.0)
    logits = jax.lax.dot_general(h, w2_ref[...], (((1,), (0,)), ((), ())),
                                 preferred_element_type=jnp.float32,
                                 precision=jax.lax.Precision.DEFAULT)
    logits = logits + b2_ref[...]
    logits_ref[...] = logits

    gates = jax.nn.sigmoid(logits)
    pos = gates > THRESHOLD

    bits = jax.lax.bitcast_convert_type(gates, jnp.int32)
    eidx = jax.lax.broadcasted_iota(jnp.int32, gates.shape, 1)
    key = ((bits - _BASE_BITS) << 6) | ((E - 1) - eidx)
    keyf = jax.lax.bitcast_convert_type(key, jnp.float32)
    cur = jnp.where(pos, keyf, 0.0)
    sel = jnp.zeros(gates.shape, dtype=jnp.bool_)
    for _ in range(TOP_K):
        m = jnp.max(cur, axis=1, keepdims=True)
        hit = cur == m
        sel = sel | hit
        cur = jnp.where(hit, 0.0, cur)
    maskf = jnp.where(sel & pos, 1.0, 0.0)
    mask_ref[...] = maskf

    w = gates * maskf
    normw_ref[...] = w / (jnp.sum(w, axis=1, keepdims=True) + 1e-6)

    mx = jnp.max(logits, axis=1, keepdims=True)
    ex = jnp.exp(logits - mx)
    probs = ex / jnp.sum(ex, axis=1, keepdims=True)

    mm = jnp.sum(maskf, axis=0, keepdims=True)
    pm = jnp.sum(probs, axis=0, keepdims=True)

    @pl.when(i == 0)
    def _init():
        acc_mask[...] = mm
        acc_prob[...] = pm

    @pl.when(i > 0)
    def _acc():
        acc_mask[...] += mm
        acc_prob[...] += pm

    @pl.when(i == n - 1)
    def _fin():
        t = jnp.float32(n * TBLK)
        s = jnp.sum(acc_mask[...] * acc_prob[...], axis=1, keepdims=True)
        loss_ref[...] = s * jnp.float32(E) / (t * t)


def kernel(hidden_states, W1, b1, W2, b2):
    B, S, H = hidden_states.shape
    T = B * S
    x = hidden_states.reshape(T, H)
    w1t = W1.T.astype(jnp.bfloat16)
    w2t = W2.T
    grid = (T // TBLK,)

    out_shape = [
        jax.ShapeDtypeStruct((T, E), jnp.float32),
        jax.ShapeDtypeStruct((T, E), jnp.float32),
        jax.ShapeDtypeStruct((T, E), jnp.float32),
        jax.ShapeDtypeStruct((1, 1), jnp.float32),
    ]
    in_specs = [
        pl.BlockSpec((TBLK, H), lambda i: (i, 0)),
        pl.BlockSpec(memory_space=pltpu.MemorySpace.HBM),
        pl.BlockSpec((1, FF), lambda i: (0, 0)),
        pl.BlockSpec((FF, E), lambda i: (0, 0)),
        pl.BlockSpec((1, E), lambda i: (0, 0)),
    ]
    out_specs = [
        pl.BlockSpec((TBLK, E), lambda i: (i, 0)),
        pl.BlockSpec((TBLK, E), lambda i: (i, 0)),
        pl.BlockSpec((TBLK, E), lambda i: (i, 0)),
        pl.BlockSpec((1, 1), lambda i: (0, 0)),
    ]
    logits, maskf, normw, loss = pl.pallas_call(
        _router_kernel,
        grid=grid,
        in_specs=in_specs,
        out_specs=out_specs,
        out_shape=out_shape,
        scratch_shapes=[pltpu.VMEM((HIDDEN, FF), jnp.bfloat16),
                        pltpu.VMEM((1, E), jnp.float32),
                        pltpu.VMEM((1, E), jnp.float32),
                        pltpu.SemaphoreType.DMA],
        compiler_params=pltpu.CompilerParams(
            dimension_semantics=("arbitrary",)),
    )(x, w1t, b1.reshape(1, FF), w2t, b2.reshape(1, E))

    dispatch_mask = maskf.astype(bool).reshape(B, S, E)
    normalized_weights = normw.reshape(B, S, E)
    router_logits = logits.reshape(B, S, E)
    return dispatch_mask, normalized_weights, loss[0, 0], router_logits


# speedup vs baseline: 1.1282x; 1.0344x over previous
import jax
import jax.numpy as jnp
from jax.experimental import pallas as pl
from jax.experimental.pallas import tpu as pltpu

HIDDEN = 4096
FF = 2048
E = 64
TOP_K = 8
THRESHOLD = 0.2
TBLK = 512
NCH = 4
CH = TBLK // NCH
FFC = FF // NCH
_BASE_BITS = 1045220557


def _epilogue_chunk(logits_p):
    gates = jax.nn.sigmoid(logits_p)
    pos = gates > THRESHOLD

    bits = jax.lax.bitcast_convert_type(gates, jnp.int32)
    eidx = jax.lax.broadcasted_iota(jnp.int32, gates.shape, 1)
    key = ((bits - _BASE_BITS) << 6) | ((E - 1) - eidx)
    keyf = jax.lax.bitcast_convert_type(key, jnp.float32)
    cur = jnp.where(pos, keyf, 0.0)
    sel = jnp.zeros(gates.shape, dtype=jnp.bool_)
    for _ in range(TOP_K):
        m = jnp.max(cur, axis=1, keepdims=True)
        hit = cur == m
        sel = sel | hit
        cur = jnp.where(hit, 0.0, cur)
    maskf = jnp.where(sel & pos, 1.0, 0.0)

    w = gates * maskf
    normw = w / (jnp.sum(w, axis=1, keepdims=True) + 1e-6)

    mx = jnp.max(logits_p, axis=1, keepdims=True)
    ex = jnp.exp(logits_p - mx)
    probs = ex / jnp.sum(ex, axis=1, keepdims=True)

    mm = jnp.sum(maskf, axis=0, keepdims=True)
    pm = jnp.sum(probs, axis=0, keepdims=True)
    return maskf, normw, mm, pm


def _router_kernel(x_ref, w1_hbm, b1_ref, w2_ref, b2_ref,
                   logits_ref, mask_ref, normw_ref, loss_ref,
                   w1_ref, lbuf, acc_mask, acc_prob, w1_sem):
    i = pl.program_id(0)
    n = pl.num_programs(0)

    @pl.when(i == 0)
    def _load_w1():
        pltpu.make_async_copy(w1_hbm, w1_ref, w1_sem).start()
        pltpu.make_async_copy(w1_hbm, w1_ref, w1_sem).wait()

    x = x_ref[...].astype(jnp.bfloat16)

    h_parts = []
    mm_tot = None
    pm_tot = None
    for k in range(NCH):
        w1c = w1_ref[:, k * FFC:(k + 1) * FFC]
        hk = jax.lax.dot_general(x, w1c, (((1,), (0,)), ((), ())),
                                 preferred_element_type=jnp.float32,
                                 precision=jax.lax.Precision.DEFAULT)
        hk = jnp.maximum(hk + b1_ref[:, k * FFC:(k + 1) * FFC], 0.0)
        h_parts.append(hk)

        lp = lbuf[pl.ds(k * CH, CH), :]
        maskf, normw, mm, pm = _epilogue_chunk(lp)
        mask_ref[pl.ds(k * CH, CH), :] = maskf
        normw_ref[pl.ds(k * CH, CH), :] = normw
        mm_tot = mm if mm_tot is None else mm_tot + mm
        pm_tot = pm if pm_tot is None else pm_tot + pm

    h = jnp.concatenate(h_parts, axis=1)
    logits = jax.lax.dot_general(h, w2_ref[...], (((1,), (0,)), ((), ())),
                                 preferred_element_type=jnp.float32,
                                 precision=jax.lax.Precision.DEFAULT)
    logits = logits + b2_ref[...]
    logits_ref[...] = logits
    lbuf[...] = logits

    @pl.when(i == 1)
    def _init():
        acc_mask[...] = mm_tot
        acc_prob[...] = pm_tot

    @pl.when(i > 1)
    def _acc():
        acc_mask[...] += mm_tot
        acc_prob[...] += pm_tot

    @pl.when(i == n - 1)
    def _fin():
        t = jnp.float32((n - 1) * TBLK)
        s = jnp.sum(acc_mask[...] * acc_prob[...], axis=1, keepdims=True)
        loss_ref[...] = s * jnp.float32(E) / (t * t)


def kernel(hidden_states, W1, b1, W2, b2):
    B, S, H = hidden_states.shape
    T = B * S
    x = hidden_states.reshape(T, H)
    w1t = W1.T.astype(jnp.bfloat16)
    w2t = W2.T
    nblk = T // TBLK
    grid = (nblk + 1,)
    last = nblk - 1

    out_shape = [
        jax.ShapeDtypeStruct((T, E), jnp.float32),
        jax.ShapeDtypeStruct((T, E), jnp.float32),
        jax.ShapeDtypeStruct((T, E), jnp.float32),
        jax.ShapeDtypeStruct((1, 1), jnp.float32),
    ]
    in_specs = [
        pl.BlockSpec((TBLK, H), lambda i: (jnp.minimum(i, last), 0)),
        pl.BlockSpec(memory_space=pltpu.MemorySpace.HBM),
        pl.BlockSpec((1, FF), lambda i: (0, 0)),
        pl.BlockSpec((FF, E), lambda i: (0, 0)),
        pl.BlockSpec((1, E), lambda i: (0, 0)),
    ]
    out_specs = [
        pl.BlockSpec((TBLK, E), lambda i: (jnp.minimum(i, last), 0)),
        pl.BlockSpec((TBLK, E), lambda i: (jnp.maximum(i - 1, 0), 0)),
        pl.BlockSpec((TBLK, E), lambda i: (jnp.maximum(i - 1, 0), 0)),
        pl.BlockSpec((1, 1), lambda i: (0, 0)),
    ]
    logits, maskf, normw, loss = pl.pallas_call(
        _router_kernel,
        grid=grid,
        in_specs=in_specs,
        out_specs=out_specs,
        out_shape=out_shape,
        scratch_shapes=[pltpu.VMEM((HIDDEN, FF), jnp.bfloat16),
                        pltpu.VMEM((TBLK, E), jnp.float32),
                        pltpu.VMEM((1, E), jnp.float32),
                        pltpu.VMEM((1, E), jnp.float32),
                        pltpu.SemaphoreType.DMA],
        compiler_params=pltpu.CompilerParams(
            dimension_semantics=("arbitrary",)),
    )(x, w1t, b1.reshape(1, FF), w2t, b2.reshape(1, E))

    dispatch_mask = maskf.astype(bool).reshape(B, S, E)
    normalized_weights = normw.reshape(B, S, E)
    router_logits = logits.reshape(B, S, E)
    return dispatch_mask, normalized_weights, loss[0, 0], router_logits
